# baseline (speedup 1.0000x reference)
import functools

import jax
import jax.numpy as jnp
from jax import lax
from jax.experimental import pallas as pl
from jax.experimental.pallas import tpu as pltpu
from jax.experimental.pallas import tpu_sc as plsc

B = 16384
F = 26
V = 1000
D = 64
NUM = 13
H = 1024
FD = F * D
SEG = F // 2
R = B * F

CH = 128
K = 4
NBUF = 2


def _sc_gather(table_flat, flat_idx, n_rows):
    info = plsc.get_sparse_core_info()
    NC, NS = info.num_cores, info.num_subcores
    NW = NC * NS
    r_per_w = n_rows // NW
    n_ch = r_per_w // CH
    NG = n_ch // K
    mesh = plsc.VectorSubcoreMesh(core_axis_name="c", subcore_axis_name="s")

    @functools.partial(
        pl.kernel,
        mesh=mesh,
        compiler_params=pltpu.CompilerParams(use_tc_tiling_on_sc=False),
        out_type=jax.ShapeDtypeStruct((n_rows, D), jnp.float32),
        scratch_types=[
            pltpu.VMEM((r_per_w,), jnp.int32),
            pltpu.VMEM((NBUF * K, CH, D), jnp.float32),
            pltpu.SemaphoreType.DMA,
            pltpu.SemaphoreType.DMA,
        ],
    )
    def k(table_hbm, idx_hbm, out_hbm, idx_v, rows_v, sem_g, sem_o):
        wid = lax.axis_index("s") * NC + lax.axis_index("c")
        base = wid * r_per_w
        pltpu.sync_copy(idx_hbm.at[pl.ds(base, r_per_w)], idx_v)

        def fire_gathers(g):
            cps = []
            for j in range(K):
                c = g * K + j
                slot = (g % NBUF) * K + j
                cps.append(pltpu.async_copy(
                    table_hbm.at[idx_v.at[pl.ds(c * CH, CH)]],
                    rows_v.at[slot], sem_g))
            return cps

        def fire_scatters(g):
            cps = []
            for j in range(K):
                c = g * K + j
                slot = (g % NBUF) * K + j
                cps.append(pltpu.async_copy(
                    rows_v.at[slot],
                    out_hbm.at[pl.ds(base + c * CH, CH)], sem_o))
            return cps

        pend_g = fire_gathers(0)
        pend_o = {}
        for g in range(NG):
            if g + 1 < NG:
                if g >= 1:
                    for cp in pend_o.pop(g - 1):
                        cp.wait()
                nxt = fire_gathers(g + 1)
            for cp in pend_g:
                cp.wait()
            pend_o[g] = fire_scatters(g)
            if g + 1 < NG:
                pend_g = nxt
        for gg, cps in pend_o.items():
            for cp in cps:
                cp.wait()

    return k(table_flat, flat_idx)


def _mlp_body(emb_ref, num_ref, mean_ref, std_ref, w1e_ref, w1n_ref,
              b1_ref, w2_ref, b2_ref, out_ref):
    e = jnp.concatenate([emb_ref[s] for s in range(SEG)], axis=1)
    x = jnp.dot(e.astype(jnp.bfloat16), w1e_ref[...],
                preferred_element_type=jnp.float32)
    numn = (num_ref[...] - mean_ref[...]) / std_ref[...]
    x = x + jnp.dot(numn, w1n_ref[...], preferred_element_type=jnp.float32)
    x = jnp.maximum(x + b1_ref[...], 0.0)
    out_ref[...] = (jnp.sum(x * w2_ref[...], axis=1, keepdims=True)
                    + b2_ref[...])


def _tc_mlp(emb3, numericals, norm_mean, norm_std, w1e_bf, w1n, b1, w2t, b2):
    BB = 512
    nb = emb3.shape[1]
    grid = (nb // BB,)
    return pl.pallas_call(
        _mlp_body,
        grid=grid,
        in_specs=[
            pl.BlockSpec((SEG, BB, 128), lambda i: (0, i, 0)),
            pl.BlockSpec((BB, NUM), lambda i: (i, 0)),
            pl.BlockSpec((1, NUM), lambda i: (0, 0)),
            pl.BlockSpec((1, NUM), lambda i: (0, 0)),
            pl.BlockSpec((FD, H), lambda i: (0, 0)),
            pl.BlockSpec((NUM, H), lambda i: (0, 0)),
            pl.BlockSpec((1, H), lambda i: (0, 0)),
            pl.BlockSpec((1, H), lambda i: (0, 0)),
            pl.BlockSpec((1, 1), lambda i: (0, 0)),
        ],
        out_specs=pl.BlockSpec((BB, 1), lambda i: (i, 0)),
        out_shape=jax.ShapeDtypeStruct((nb, 1), jnp.float32),
    )(emb3, numericals, norm_mean, norm_std, w1e_bf, w1n, b1, w2t, b2)


def kernel(cat_indices, numericals, emb_tables, norm_mean, norm_std,
           W1, b1, W2, b2):
    fidx = (cat_indices.astype(jnp.int32)
            + (jnp.arange(F, dtype=jnp.int32) * V)[None, :])
    idx3 = fidx.reshape(B, SEG, 2).transpose(1, 0, 2)
    table_flat = emb_tables.reshape(F * V, D)

    w1e_bf = W1[:FD].astype(jnp.bfloat16)
    w1n = W1[FD:]
    mean2 = norm_mean.reshape(1, NUM)
    std2 = norm_std.reshape(1, NUM)
    b1r = b1.reshape(1, H)
    w2t = W2.reshape(1, H)
    b2r = b2.reshape(1, 1)

    HB = B // 2
    outs = []
    rows_h = []
    for h in range(2):
        idx_h = idx3[:, h * HB:(h + 1) * HB, :].reshape(HB * F)
        rows_h.append(_sc_gather(table_flat, idx_h, HB * F))
    for h in range(2):
        emb3 = rows_h[h].reshape(SEG, HB, 128)
        num_h = numericals[h * HB:(h + 1) * HB]
        outs.append(_tc_mlp(emb3, num_h, mean2, std2,
                            w1e_bf, w1n, b1r, w2t, b2r))
    return jnp.concatenate(outs, axis=0)


# speedup vs baseline: 1.1662x; 1.1662x over previous
import functools

import jax
import jax.numpy as jnp
from jax import lax
from jax.experimental import pallas as pl
from jax.experimental.pallas import tpu as pltpu
from jax.experimental.pallas import tpu_sc as plsc

B = 16384
F = 26
V = 1000
D = 64
NUM = 13
H = 1024
FD = F * D
SEG = F // 2
R = B * F

CH = 128
K = 4
NBUF = 2


def _sc_gather(table_flat, flat_idx):
    info = plsc.get_sparse_core_info()
    NC, NS = info.num_cores, info.num_subcores
    NW = NC * NS
    r_per_w = R // NW
    n_ch = r_per_w // CH
    NG = n_ch // K
    mesh = plsc.VectorSubcoreMesh(core_axis_name="c", subcore_axis_name="s")

    @functools.partial(
        pl.kernel,
        mesh=mesh,
        compiler_params=pltpu.CompilerParams(use_tc_tiling_on_sc=False),
        out_type=jax.ShapeDtypeStruct((R, D), jnp.float32),
        scratch_types=[
            pltpu.VMEM((r_per_w,), jnp.int32),
            pltpu.VMEM((NBUF * K, CH, D), jnp.float32),
            pltpu.SemaphoreType.DMA,
            pltpu.SemaphoreType.DMA,
        ],
    )
    def k(table_hbm, idx_hbm, out_hbm, idx_v, rows_v, sem_g, sem_o):
        wid = lax.axis_index("s") * NC + lax.axis_index("c")
        base = wid * r_per_w
        pltpu.sync_copy(idx_hbm.at[pl.ds(base, r_per_w)], idx_v)

        def fire_gathers(g):
            cps = []
            for j in range(K):
                c = g * K + j
                slot = (g % NBUF) * K + j
                cps.append(pltpu.async_copy(
                    table_hbm.at[idx_v.at[pl.ds(c * CH, CH)]],
                    rows_v.at[slot], sem_g))
            return cps

        def fire_scatters(g):
            cps = []
            for j in range(K):
                c = g * K + j
                slot = (g % NBUF) * K + j
                cps.append(pltpu.async_copy(
                    rows_v.at[slot],
                    out_hbm.at[pl.ds(base + c * CH, CH)], sem_o))
            return cps

        pend_g = fire_gathers(0)
        pend_o = {}
        for g in range(NG):
            if g + 1 < NG:
                if g >= 1:
                    for cp in pend_o.pop(g - 1):
                        cp.wait()
                nxt = fire_gathers(g + 1)
            for cp in pend_g:
                cp.wait()
            pend_o[g] = fire_scatters(g)
            if g + 1 < NG:
                pend_g = nxt
        for gg, cps in pend_o.items():
            for cp in cps:
                cp.wait()

    return k(table_flat, flat_idx)


def _mlp_body(emb_ref, num_ref, mean_ref, std_ref, w1e_ref, w1n_ref,
              b1_ref, w2_ref, b2_ref, out_ref):
    e = jnp.concatenate([emb_ref[s] for s in range(SEG)], axis=1)
    x = jnp.dot(e.astype(jnp.bfloat16), w1e_ref[...],
                preferred_element_type=jnp.float32)
    numn = (num_ref[...] - mean_ref[...]) / std_ref[...]
    x = x + jnp.dot(numn, w1n_ref[...], preferred_element_type=jnp.float32)
    x = jnp.maximum(x + b1_ref[...], 0.0)
    out_ref[...] = (jnp.sum(x * w2_ref[...], axis=1, keepdims=True)
                    + b2_ref[...])


def _tc_mlp(emb3, numericals, norm_mean, norm_std, w1e_bf, w1n, b1, w2t, b2):
    BB = 1024
    grid = (B // BB,)
    return pl.pallas_call(
        _mlp_body,
        grid=grid,
        in_specs=[
            pl.BlockSpec((SEG, BB, 128), lambda i: (0, i, 0)),
            pl.BlockSpec((BB, NUM), lambda i: (i, 0)),
            pl.BlockSpec((1, NUM), lambda i: (0, 0)),
            pl.BlockSpec((1, NUM), lambda i: (0, 0)),
            pl.BlockSpec((FD, H), lambda i: (0, 0)),
            pl.BlockSpec((NUM, H), lambda i: (0, 0)),
            pl.BlockSpec((1, H), lambda i: (0, 0)),
            pl.BlockSpec((1, H), lambda i: (0, 0)),
            pl.BlockSpec((1, 1), lambda i: (0, 0)),
        ],
        out_specs=pl.BlockSpec((BB, 1), lambda i: (i, 0)),
        out_shape=jax.ShapeDtypeStruct((B, 1), jnp.float32),
    )(emb3, numericals, norm_mean, norm_std, w1e_bf, w1n, b1, w2t, b2)


def kernel(cat_indices, numericals, emb_tables, norm_mean, norm_std,
           W1, b1, W2, b2):
    fidx = (cat_indices.astype(jnp.int32)
            + (jnp.arange(F, dtype=jnp.int32) * V)[None, :])
    flat_idx = (fidx.reshape(B, SEG, 2)
                .transpose(1, 0, 2)
                .reshape(R))
    table_flat = emb_tables.reshape(F * V, D)

    rows = _sc_gather(table_flat, flat_idx)
    emb3 = rows.reshape(SEG, B, 128)

    w1e_bf = W1[:FD].astype(jnp.bfloat16)
    w1n = W1[FD:]
    out = _tc_mlp(emb3, numericals,
                  norm_mean.reshape(1, NUM), norm_std.reshape(1, NUM),
                  w1e_bf, w1n, b1.reshape(1, H), W2.reshape(1, H),
                  b2.reshape(1, 1))
    return out
